# Initial kernel scaffold; baseline (speedup 1.0000x reference)
import jax
import jax.numpy as jnp
from jax.experimental import pallas as pl


def kernel(x, edge_index, edge_weight, W1, b1, W_out, b_out, W_root):
    raise NotImplementedError("write your pallas kernel here")


# speedup vs baseline: 9.1227x; 9.1227x over previous
import functools

import jax
import jax.numpy as jnp
from jax import lax
from jax.experimental import pallas as pl
from jax.experimental.pallas import tpu as pltpu
from jax.experimental.pallas import tpu_sc as plsc

N = 10000
E = 320000
D = 128
HD = D // 2
NC = 2
NS = 16
NWK = NC * NS
K = 128
CS = 8

EP = 327680
W1N = EP // NWK // K
C1N = W1N // CS
W2N = EP // NS // K
C2N = W2N // CS

NZC = N // K
NZR = N - NZC * K

_mesh = plsc.VectorSubcoreMesh(core_axis_name="c", subcore_axis_name="s",
                               num_cores=NC, num_subcores=NS)
f32 = jnp.float32


def _zero16():
    return jnp.zeros((16,), f32)


@functools.partial(
    pl.kernel,
    out_type=jax.ShapeDtypeStruct((NC, 2, N), f32),
    mesh=_mesh,
    scratch_types=[
        pltpu.VMEM((CS, K), jnp.int32),
        pltpu.VMEM((CS, K), jnp.int32),
        pltpu.VMEM((CS, K), f32),
        pltpu.VMEM((CS, K), f32),
        pltpu.VMEM((2000,), f32),
        pltpu.VMEM_SHARED((N,), f32),
        pltpu.VMEM_SHARED((N,), f32),
    ],
)
def _sc_degrees(row3, col3, w3, parts, row_v, col_v, w_v, val_v, zz,
                deg_sp, cnt_sp):
    c = lax.axis_index("c")
    s = lax.axis_index("s")
    wid = s * NC + c

    @pl.when(s == 0)
    def _zero():
        def zb(i, _):
            zz[pl.ds(i * 16, 16)] = _zero16()
            return ()
        lax.fori_loop(0, 125, zb, ())

        def cb(j, _):
            pltpu.sync_copy(zz, deg_sp.at[pl.ds(j * 2000, 2000)])
            pltpu.sync_copy(zz, cnt_sp.at[pl.ds(j * 2000, 2000)])
            return ()
        lax.fori_loop(0, 5, cb, ())

    plsc.subcore_barrier()

    one = jnp.ones((16,), f32)
    zero = _zero16()

    def cb(ch, _):
        pltpu.sync_copy(row3.at[wid, pl.ds(ch * CS, CS)], row_v)
        pltpu.sync_copy(col3.at[wid, pl.ds(ch * CS, CS)], col_v)
        pltpu.sync_copy(w3.at[wid, pl.ds(ch * CS, CS)], w_v)

        def wb(w8, _):
            for j in range(K // 16):
                rv = row_v[w8, pl.ds(j * 16, 16)]
                cv = col_v[w8, pl.ds(j * 16, 16)]
                val_v[w8, pl.ds(j * 16, 16)] = jnp.where(rv != cv, one, zero)
            pltpu.sync_copy(w_v.at[w8], deg_sp.at[col_v.at[w8]], add=True)
            pltpu.sync_copy(val_v.at[w8], cnt_sp.at[col_v.at[w8]], add=True)
            return ()
        lax.fori_loop(0, CS, wb, ())
        return ()
    lax.fori_loop(0, C1N, cb, ())

    plsc.subcore_barrier()

    @pl.when(s == 0)
    def _out():
        pltpu.sync_copy(deg_sp, parts.at[c, 0])
        pltpu.sync_copy(cnt_sp, parts.at[c, 1])


HN = N // NC
AR = HN + 8
AZC = AR // K
AZR = AR - AZC * K
ADR = HN - AZC * K

_AGG_KW = dict(
    out_type=jax.ShapeDtypeStruct((N, D), f32),
    mesh=_mesh,
    scratch_types=[
        pltpu.VMEM((CS, K), jnp.int32),
        pltpu.VMEM((CS, K), jnp.int32),
        pltpu.VMEM((CS, K), f32),
        pltpu.VMEM((K, D), f32),
        pltpu.VMEM((K, D), f32),
        pltpu.VMEM_SHARED((AR, D), f32),
        pltpu.SemaphoreType.DMA,
    ],
)


def _sc_aggregate_body(hs, row2, col2, w2, accp, row_v, col_v, w_v, rows_b,
                       sc_b, acc_sp, sem):
    c = lax.axis_index("c")
    s = lax.axis_index("s")

    def zb(i, _):
        for j in range(D // 16):
            sc_b[i, pl.ds(j * 16, 16)] = _zero16()
        return ()
    lax.fori_loop(0, K, zb, ())

    def zc(k, _):
        cid = k * NS + s

        @pl.when(cid < AZC)
        def _full():
            pltpu.sync_copy(sc_b, acc_sp.at[pl.ds(cid * K, K)])

        @pl.when(cid == AZC)
        def _rem():
            pltpu.sync_copy(sc_b.at[pl.ds(0, AZR)],
                            acc_sp.at[pl.ds(AZC * K, AZR)])
        return ()
    lax.fori_loop(0, 3, zc, ())

    plsc.subcore_barrier()

    nbase = c * HN

    def cb(ch, _):
        pltpu.sync_copy(row2.at[s, pl.ds(ch * CS, CS)], row_v)
        pltpu.sync_copy(col2.at[s, pl.ds(ch * CS, CS)], col_v)
        pltpu.sync_copy(w2.at[s, pl.ds(ch * CS, CS)], w_v)

        def lb(w8, _):
            for j in range(K // 16):
                sl = pl.ds(j * 16, 16)
                loc = col_v[w8, sl] - nbase
                oob = (loc < 0) | (loc >= HN)
                col_v[w8, sl] = jnp.where(oob, HN, loc)
            return ()
        lax.fori_loop(0, CS, lb, ())

        def wb(w8, _):
            pltpu.async_copy(hs.at[row_v.at[w8]], rows_b, sem).wait()

            def gb(g, _):
                wv = w_v[w8, pl.ds(g * 16, 16)]
                for l in range(16):
                    r = g * 16 + l
                    spl = jnp.broadcast_to(wv[l], (16,))
                    for cc in range(D // 16):
                        sc_b[r, pl.ds(cc * 16, 16)] = (
                            rows_b[r, pl.ds(cc * 16, 16)] * spl)
                return ()
            lax.fori_loop(0, K // 16, gb, ())
            pltpu.sync_copy(sc_b, acc_sp.at[col_v.at[w8]], add=True)
            return ()
        lax.fori_loop(0, CS, wb, ())
        return ()
    lax.fori_loop(0, C2N, cb, ())

    plsc.subcore_barrier()

    def oc(k, _):
        cid = k * NS + s

        @pl.when(cid < AZC)
        def _full():
            pltpu.sync_copy(acc_sp.at[pl.ds(cid * K, K)],
                            accp.at[pl.ds(nbase + cid * K, K)])

        @pl.when(cid == AZC)
        def _rem():
            pltpu.sync_copy(acc_sp.at[pl.ds(AZC * K, ADR)],
                            accp.at[pl.ds(nbase + AZC * K, ADR)])
        return ()
    lax.fori_loop(0, 3, oc, ())


_sc_aggregate = pl.kernel(_sc_aggregate_body, **_AGG_KW)


@functools.partial(
    pl.kernel,
    out_type=jax.ShapeDtypeStruct((NC, N), f32),
    mesh=_mesh,
    scratch_types=[
        pltpu.VMEM((CS, K), jnp.int32),
        pltpu.VMEM((CS, K), jnp.int32),
        pltpu.VMEM((K,), f32),
        pltpu.VMEM((CS, K), f32),
        pltpu.VMEM((2000,), f32),
        pltpu.VMEM_SHARED((N,), f32),
        pltpu.SemaphoreType.DMA,
    ],
)
def _sc_scalar_agg(y1, row3, col3, zp, row_v, col_v, yv_b, val_v, zz,
                   z_sp, sem):
    c = lax.axis_index("c")
    s = lax.axis_index("s")
    wid = s * NC + c

    @pl.when(s == 0)
    def _zero():
        def zb(i, _):
            zz[pl.ds(i * 16, 16)] = _zero16()
            return ()
        lax.fori_loop(0, 125, zb, ())

        def cb(j, _):
            pltpu.sync_copy(zz, z_sp.at[pl.ds(j * 2000, 2000)])
            return ()
        lax.fori_loop(0, 5, cb, ())

    plsc.subcore_barrier()

    zero = _zero16()

    def cb(ch, _):
        pltpu.sync_copy(row3.at[wid, pl.ds(ch * CS, CS)], row_v)
        pltpu.sync_copy(col3.at[wid, pl.ds(ch * CS, CS)], col_v)

        def wb(w8, _):
            pltpu.async_copy(y1.at[row_v.at[w8]], yv_b, sem).wait()
            for j in range(K // 16):
                rv = row_v[w8, pl.ds(j * 16, 16)]
                cv = col_v[w8, pl.ds(j * 16, 16)]
                yv = yv_b[pl.ds(j * 16, 16)]
                val_v[w8, pl.ds(j * 16, 16)] = jnp.where(rv != cv, yv, zero)
            pltpu.sync_copy(val_v.at[w8], z_sp.at[col_v.at[w8]], add=True)
            return ()
        lax.fori_loop(0, CS, wb, ())
        return ()
    lax.fori_loop(0, C1N, cb, ())

    plsc.subcore_barrier()

    @pl.when(s == 0)
    def _out():
        pltpu.sync_copy(z_sp, zp.at[c])


RB = 2000
GRID = N // RB


def _tc_ab_body(x_r, w1_r, d0_r, d1_r, h0_r, hs_r):
    h0 = jnp.dot(x_r[...], w1_r[...], preferred_element_type=f32)
    deg = d0_r[...] + d1_r[...] + 1.0
    h0_r[...] = h0
    hs_r[...] = h0 * lax.rsqrt(deg)


def _tc_ab(x, w1, d0, d1):
    return pl.pallas_call(
        _tc_ab_body,
        grid=(GRID,),
        in_specs=[
            pl.BlockSpec((RB, D), lambda i: (i, 0)),
            pl.BlockSpec((D, D), lambda i: (0, 0)),
            pl.BlockSpec((RB, 1), lambda i: (i, 0)),
            pl.BlockSpec((RB, 1), lambda i: (i, 0)),
        ],
        out_specs=[
            pl.BlockSpec((RB, D), lambda i: (i, 0)),
            pl.BlockSpec((RB, D), lambda i: (i, 0)),
        ],
        out_shape=[
            jax.ShapeDtypeStruct((N, D), f32),
            jax.ShapeDtypeStruct((N, D), f32),
        ],
    )(x, w1, d0, d1)


def _tc_c_body(acc_r, h0_r, d0_r, d1_r, b1_r, wo_r, wr_r, bo_r,
               feat_r, y_r, rp_r):
    deg = d0_r[...] + d1_r[...] + 1.0
    dis = lax.rsqrt(deg)
    h = dis * acc_r[...] + h0_r[...] / deg + b1_r[...]
    h = jnp.maximum(h, 0.0)
    feat_r[...] = h
    y_r[...] = jnp.dot(h, wo_r[...], preferred_element_type=f32)
    rp_r[...] = (jnp.dot(h, wr_r[...], preferred_element_type=f32)
                 + bo_r[...])


def _tc_c(accp, h0, d0, d1, b1, wo, wr, bo):
    return pl.pallas_call(
        _tc_c_body,
        grid=(GRID,),
        in_specs=[
            pl.BlockSpec((RB, D), lambda i: (i, 0)),
            pl.BlockSpec((RB, D), lambda i: (i, 0)),
            pl.BlockSpec((RB, 1), lambda i: (i, 0)),
            pl.BlockSpec((RB, 1), lambda i: (i, 0)),
            pl.BlockSpec((1, D), lambda i: (0, 0)),
            pl.BlockSpec((D, 1), lambda i: (0, 0)),
            pl.BlockSpec((D, 1), lambda i: (0, 0)),
            pl.BlockSpec((1, 1), lambda i: (0, 0)),
        ],
        out_specs=[
            pl.BlockSpec((RB, D), lambda i: (i, 0)),
            pl.BlockSpec((RB, 1), lambda i: (i, 0)),
            pl.BlockSpec((RB, 1), lambda i: (i, 0)),
        ],
        out_shape=[
            jax.ShapeDtypeStruct((N, D), f32),
            jax.ShapeDtypeStruct((N, 1), f32),
            jax.ShapeDtypeStruct((N, 1), f32),
        ],
    )(accp, h0, d0, d1, b1, wo, wr, bo)


def _tc_d_body(z0_r, z1_r, y_r, rp_r, c0_r, c1_r, o_r):
    cnt = c0_r[...] + c1_r[...] + 1.0
    o_r[...] = (z0_r[...] + z1_r[...] + y_r[...]) / jnp.maximum(cnt, 1.0) \
        + rp_r[...]


def _tc_d(z0, z1, y, rp, c0, c1):
    return pl.pallas_call(
        _tc_d_body,
        grid=(GRID,),
        in_specs=[pl.BlockSpec((RB, 1), lambda i: (i, 0))] * 6,
        out_specs=pl.BlockSpec((RB, 1), lambda i: (i, 0)),
        out_shape=jax.ShapeDtypeStruct((N, 1), f32),
    )(z0, z1, y, rp, c0, c1)


def kernel(x, edge_index, edge_weight, W1, b1, W_out, b_out, W_root):
    pad = EP - E
    rowp = jnp.concatenate([edge_index[0],
                            jnp.zeros((pad,), jnp.int32)])
    colp = jnp.concatenate([edge_index[1],
                            jnp.zeros((pad,), jnp.int32)])
    wp = jnp.concatenate([edge_weight, jnp.zeros((pad,), f32)])

    row3 = rowp.reshape(NWK, W1N, K)
    col3 = colp.reshape(NWK, W1N, K)
    w3 = wp.reshape(NWK, W1N, K)
    row2 = rowp.reshape(NS, W2N, K)
    col2 = colp.reshape(NS, W2N, K)
    w2 = wp.reshape(NS, W2N, K)

    parts = _sc_degrees(row3, col3, w3)
    d0 = parts[0, 0].reshape(N, 1)
    d1 = parts[1, 0].reshape(N, 1)
    c0 = parts[0, 1].reshape(N, 1)
    c1 = parts[1, 1].reshape(N, 1)

    h0, hs = _tc_ab(x, W1, d0, d1)

    accp = _sc_aggregate(hs, row2, col2, w2)

    feat, y, rp = _tc_c(accp, h0, d0, d1,
                        b1.reshape(1, D), W_out, W_root,
                        b_out.reshape(1, 1))

    zp = _sc_scalar_agg(y.reshape(N), row3, col3)

    out2 = _tc_d(zp[0].reshape(N, 1), zp[1].reshape(N, 1), y, rp,
                 c0, c1)
    return out2.reshape(-1), feat


# speedup vs baseline: 10.1847x; 1.1164x over previous
import functools

import jax
import jax.numpy as jnp
from jax import lax
from jax.experimental import pallas as pl
from jax.experimental.pallas import tpu as pltpu
from jax.experimental.pallas import tpu_sc as plsc

N = 10000
E = 320000
D = 128
HD = D // 2
NC = 2
NS = 16
NWK = NC * NS
K = 128
CS = 8

EP = 327680
W1N = EP // NWK // K
C1N = W1N // CS
W2N = EP // NS // K
C2N = W2N // CS

NZC = N // K
NZR = N - NZC * K

_mesh = plsc.VectorSubcoreMesh(core_axis_name="c", subcore_axis_name="s",
                               num_cores=NC, num_subcores=NS)
f32 = jnp.float32


def _zero16():
    return jnp.zeros((16,), f32)


@functools.partial(
    pl.kernel,
    out_type=jax.ShapeDtypeStruct((NC, 2, N), f32),
    mesh=_mesh,
    scratch_types=[
        pltpu.VMEM((CS, K), jnp.int32),
        pltpu.VMEM((CS, K), jnp.int32),
        pltpu.VMEM((CS, K), f32),
        pltpu.VMEM((CS, K), f32),
        pltpu.VMEM((2000,), f32),
        pltpu.VMEM_SHARED((N,), f32),
        pltpu.VMEM_SHARED((N,), f32),
    ],
)
def _sc_degrees(row3, col3, w3, parts, row_v, col_v, w_v, val_v, zz,
                deg_sp, cnt_sp):
    c = lax.axis_index("c")
    s = lax.axis_index("s")
    wid = s * NC + c

    @pl.when(s == 0)
    def _zero():
        def zb(i, _):
            zz[pl.ds(i * 16, 16)] = _zero16()
            return ()
        lax.fori_loop(0, 125, zb, ())

        def cb(j, _):
            pltpu.sync_copy(zz, deg_sp.at[pl.ds(j * 2000, 2000)])
            pltpu.sync_copy(zz, cnt_sp.at[pl.ds(j * 2000, 2000)])
            return ()
        lax.fori_loop(0, 5, cb, ())

    plsc.subcore_barrier()

    one = jnp.ones((16,), f32)
    zero = _zero16()

    def cb(ch, _):
        pltpu.sync_copy(row3.at[wid, pl.ds(ch * CS, CS)], row_v)
        pltpu.sync_copy(col3.at[wid, pl.ds(ch * CS, CS)], col_v)
        pltpu.sync_copy(w3.at[wid, pl.ds(ch * CS, CS)], w_v)

        def wb(w8, _):
            for j in range(K // 16):
                rv = row_v[w8, pl.ds(j * 16, 16)]
                cv = col_v[w8, pl.ds(j * 16, 16)]
                val_v[w8, pl.ds(j * 16, 16)] = jnp.where(rv != cv, one, zero)
            pltpu.sync_copy(w_v.at[w8], deg_sp.at[col_v.at[w8]], add=True)
            pltpu.sync_copy(val_v.at[w8], cnt_sp.at[col_v.at[w8]], add=True)
            return ()
        lax.fori_loop(0, CS, wb, ())
        return ()
    lax.fori_loop(0, C1N, cb, ())

    plsc.subcore_barrier()

    @pl.when(s == 0)
    def _out():
        pltpu.sync_copy(deg_sp, parts.at[c, 0])
        pltpu.sync_copy(cnt_sp, parts.at[c, 1])


HN = N // NC
AR = HN + 8
AZC = AR // K
AZR = AR - AZC * K
ADR = HN - AZC * K

_AGG_KW = dict(
    out_type=jax.ShapeDtypeStruct((N, D), f32),
    mesh=_mesh,
    scratch_types=[
        pltpu.VMEM((CS, K), jnp.int32),
        pltpu.VMEM((CS, K), jnp.int32),
        pltpu.VMEM((CS, K), f32),
        pltpu.VMEM((K, D), f32),
        pltpu.VMEM((K, D), f32),
        pltpu.VMEM((K, D), f32),
        pltpu.VMEM((K, D), f32),
        pltpu.VMEM_SHARED((AR, D), f32),
        pltpu.SemaphoreType.DMA,
        pltpu.SemaphoreType.DMA,
        pltpu.SemaphoreType.DMA,
        pltpu.SemaphoreType.DMA,
    ],
)


def _sc_aggregate_body(hs, row2, col2, w2, accp, row_v, col_v, w_v,
                       rows_b0, rows_b1, sc_b0, sc_b1, acc_sp,
                       sem_g0, sem_g1, sem_s0, sem_s1):
    c = lax.axis_index("c")
    s = lax.axis_index("s")

    def zb(i, _):
        for j in range(D // 16):
            sc_b0[i, pl.ds(j * 16, 16)] = _zero16()
        return ()
    lax.fori_loop(0, K, zb, ())

    def zc(k, _):
        cid = k * NS + s

        @pl.when(cid < AZC)
        def _full():
            pltpu.sync_copy(sc_b0, acc_sp.at[pl.ds(cid * K, K)])

        @pl.when(cid == AZC)
        def _rem():
            pltpu.sync_copy(sc_b0.at[pl.ds(0, AZR)],
                            acc_sp.at[pl.ds(AZC * K, AZR)])
        return ()
    lax.fori_loop(0, 3, zc, ())

    plsc.subcore_barrier()

    nbase = c * HN

    def scale(w8, rb, sb):
        def gb(g, _):
            wv = w_v[w8, pl.ds(g * 16, 16)]
            for l in range(16):
                r = g * 16 + l
                spl = jnp.broadcast_to(wv[l], (16,))
                for cc in range(D // 16):
                    sb[r, pl.ds(cc * 16, 16)] = (
                        rb[r, pl.ds(cc * 16, 16)] * spl)
            return ()
        lax.fori_loop(0, K // 16, gb, ())

    def cb(ch, _):
        pltpu.sync_copy(row2.at[s, pl.ds(ch * CS, CS)], row_v)
        pltpu.sync_copy(col2.at[s, pl.ds(ch * CS, CS)], col_v)
        pltpu.sync_copy(w2.at[s, pl.ds(ch * CS, CS)], w_v)

        def lb(w8, _):
            for j in range(K // 16):
                sl = pl.ds(j * 16, 16)
                loc = col_v[w8, sl] - nbase
                oob = (loc < 0) | (loc >= HN)
                col_v[w8, sl] = jnp.where(oob, HN, loc)
            return ()
        lax.fori_loop(0, CS, lb, ())

        pltpu.async_copy(hs.at[row_v.at[0]], rows_b0, sem_g0)

        def pb(p, _):
            e = 2 * p
            o = e + 1
            pltpu.make_async_copy(hs.at[row_v.at[e]], rows_b0,
                                  sem_g0).wait()
            pltpu.async_copy(hs.at[row_v.at[o]], rows_b1, sem_g1)

            @pl.when(p > 0)
            def _ws0():
                pltpu.make_async_copy(sc_b0, acc_sp.at[col_v.at[e]],
                                      sem_s0).wait()
            scale(e, rows_b0, sc_b0)
            pltpu.async_copy(sc_b0, acc_sp.at[col_v.at[e]], sem_s0,
                             add=True)
            pltpu.make_async_copy(hs.at[row_v.at[o]], rows_b1,
                                  sem_g1).wait()

            @pl.when(p < CS // 2 - 1)
            def _g0():
                pltpu.async_copy(hs.at[row_v.at[e + 2]], rows_b0, sem_g0)

            @pl.when(p > 0)
            def _ws1():
                pltpu.make_async_copy(sc_b1, acc_sp.at[col_v.at[o]],
                                      sem_s1).wait()
            scale(o, rows_b1, sc_b1)
            pltpu.async_copy(sc_b1, acc_sp.at[col_v.at[o]], sem_s1,
                             add=True)
            return ()
        lax.fori_loop(0, CS // 2, pb, ())

        pltpu.make_async_copy(sc_b0, acc_sp.at[col_v.at[CS - 2]],
                              sem_s0).wait()
        pltpu.make_async_copy(sc_b1, acc_sp.at[col_v.at[CS - 1]],
                              sem_s1).wait()
        return ()
    lax.fori_loop(0, C2N, cb, ())

    plsc.subcore_barrier()

    def oc(k, _):
        cid = k * NS + s

        @pl.when(cid < AZC)
        def _full():
            pltpu.sync_copy(acc_sp.at[pl.ds(cid * K, K)],
                            accp.at[pl.ds(nbase + cid * K, K)])

        @pl.when(cid == AZC)
        def _rem():
            pltpu.sync_copy(acc_sp.at[pl.ds(AZC * K, ADR)],
                            accp.at[pl.ds(nbase + AZC * K, ADR)])
        return ()
    lax.fori_loop(0, 3, oc, ())


_sc_aggregate = pl.kernel(_sc_aggregate_body, **_AGG_KW)


@functools.partial(
    pl.kernel,
    out_type=jax.ShapeDtypeStruct((NC, N), f32),
    mesh=_mesh,
    scratch_types=[
        pltpu.VMEM((CS, K), jnp.int32),
        pltpu.VMEM((CS, K), jnp.int32),
        pltpu.VMEM((K,), f32),
        pltpu.VMEM((CS, K), f32),
        pltpu.VMEM((2000,), f32),
        pltpu.VMEM_SHARED((N,), f32),
        pltpu.SemaphoreType.DMA,
    ],
)
def _sc_scalar_agg(y1, row3, col3, zp, row_v, col_v, yv_b, val_v, zz,
                   z_sp, sem):
    c = lax.axis_index("c")
    s = lax.axis_index("s")
    wid = s * NC + c

    @pl.when(s == 0)
    def _zero():
        def zb(i, _):
            zz[pl.ds(i * 16, 16)] = _zero16()
            return ()
        lax.fori_loop(0, 125, zb, ())

        def cb(j, _):
            pltpu.sync_copy(zz, z_sp.at[pl.ds(j * 2000, 2000)])
            return ()
        lax.fori_loop(0, 5, cb, ())

    plsc.subcore_barrier()

    zero = _zero16()

    def cb(ch, _):
        pltpu.sync_copy(row3.at[wid, pl.ds(ch * CS, CS)], row_v)
        pltpu.sync_copy(col3.at[wid, pl.ds(ch * CS, CS)], col_v)

        def wb(w8, _):
            pltpu.async_copy(y1.at[row_v.at[w8]], yv_b, sem).wait()
            for j in range(K // 16):
                rv = row_v[w8, pl.ds(j * 16, 16)]
                cv = col_v[w8, pl.ds(j * 16, 16)]
                yv = yv_b[pl.ds(j * 16, 16)]
                val_v[w8, pl.ds(j * 16, 16)] = jnp.where(rv != cv, yv, zero)
            pltpu.sync_copy(val_v.at[w8], z_sp.at[col_v.at[w8]], add=True)
            return ()
        lax.fori_loop(0, CS, wb, ())
        return ()
    lax.fori_loop(0, C1N, cb, ())

    plsc.subcore_barrier()

    @pl.when(s == 0)
    def _out():
        pltpu.sync_copy(z_sp, zp.at[c])


RB = 2000
GRID = N // RB


def _tc_ab_body(x_r, w1_r, d0_r, d1_r, h0_r, hs_r):
    h0 = jnp.dot(x_r[...], w1_r[...], preferred_element_type=f32)
    deg = d0_r[...] + d1_r[...] + 1.0
    h0_r[...] = h0
    hs_r[...] = h0 * lax.rsqrt(deg)


def _tc_ab(x, w1, d0, d1):
    return pl.pallas_call(
        _tc_ab_body,
        grid=(GRID,),
        in_specs=[
            pl.BlockSpec((RB, D), lambda i: (i, 0)),
            pl.BlockSpec((D, D), lambda i: (0, 0)),
            pl.BlockSpec((RB, 1), lambda i: (i, 0)),
            pl.BlockSpec((RB, 1), lambda i: (i, 0)),
        ],
        out_specs=[
            pl.BlockSpec((RB, D), lambda i: (i, 0)),
            pl.BlockSpec((RB, D), lambda i: (i, 0)),
        ],
        out_shape=[
            jax.ShapeDtypeStruct((N, D), f32),
            jax.ShapeDtypeStruct((N, D), f32),
        ],
    )(x, w1, d0, d1)


def _tc_c_body(acc_r, h0_r, d0_r, d1_r, b1_r, wo_r, wr_r, bo_r,
               feat_r, y_r, rp_r):
    deg = d0_r[...] + d1_r[...] + 1.0
    dis = lax.rsqrt(deg)
    h = dis * acc_r[...] + h0_r[...] / deg + b1_r[...]
    h = jnp.maximum(h, 0.0)
    feat_r[...] = h
    y_r[...] = jnp.dot(h, wo_r[...], preferred_element_type=f32)
    rp_r[...] = (jnp.dot(h, wr_r[...], preferred_element_type=f32)
                 + bo_r[...])


def _tc_c(accp, h0, d0, d1, b1, wo, wr, bo):
    return pl.pallas_call(
        _tc_c_body,
        grid=(GRID,),
        in_specs=[
            pl.BlockSpec((RB, D), lambda i: (i, 0)),
            pl.BlockSpec((RB, D), lambda i: (i, 0)),
            pl.BlockSpec((RB, 1), lambda i: (i, 0)),
            pl.BlockSpec((RB, 1), lambda i: (i, 0)),
            pl.BlockSpec((1, D), lambda i: (0, 0)),
            pl.BlockSpec((D, 1), lambda i: (0, 0)),
            pl.BlockSpec((D, 1), lambda i: (0, 0)),
            pl.BlockSpec((1, 1), lambda i: (0, 0)),
        ],
        out_specs=[
            pl.BlockSpec((RB, D), lambda i: (i, 0)),
            pl.BlockSpec((RB, 1), lambda i: (i, 0)),
            pl.BlockSpec((RB, 1), lambda i: (i, 0)),
        ],
        out_shape=[
            jax.ShapeDtypeStruct((N, D), f32),
            jax.ShapeDtypeStruct((N, 1), f32),
            jax.ShapeDtypeStruct((N, 1), f32),
        ],
    )(accp, h0, d0, d1, b1, wo, wr, bo)


def _tc_d_body(z0_r, z1_r, y_r, rp_r, c0_r, c1_r, o_r):
    cnt = c0_r[...] + c1_r[...] + 1.0
    o_r[...] = (z0_r[...] + z1_r[...] + y_r[...]) / jnp.maximum(cnt, 1.0) \
        + rp_r[...]


def _tc_d(z0, z1, y, rp, c0, c1):
    return pl.pallas_call(
        _tc_d_body,
        grid=(GRID,),
        in_specs=[pl.BlockSpec((RB, 1), lambda i: (i, 0))] * 6,
        out_specs=pl.BlockSpec((RB, 1), lambda i: (i, 0)),
        out_shape=jax.ShapeDtypeStruct((N, 1), f32),
    )(z0, z1, y, rp, c0, c1)


def kernel(x, edge_index, edge_weight, W1, b1, W_out, b_out, W_root):
    pad = EP - E
    rowp = jnp.concatenate([edge_index[0],
                            jnp.zeros((pad,), jnp.int32)])
    colp = jnp.concatenate([edge_index[1],
                            jnp.zeros((pad,), jnp.int32)])
    wp = jnp.concatenate([edge_weight, jnp.zeros((pad,), f32)])

    row3 = rowp.reshape(NWK, W1N, K)
    col3 = colp.reshape(NWK, W1N, K)
    w3 = wp.reshape(NWK, W1N, K)
    row2 = rowp.reshape(NS, W2N, K)
    col2 = colp.reshape(NS, W2N, K)
    w2 = wp.reshape(NS, W2N, K)

    parts = _sc_degrees(row3, col3, w3)
    d0 = parts[0, 0].reshape(N, 1)
    d1 = parts[1, 0].reshape(N, 1)
    c0 = parts[0, 1].reshape(N, 1)
    c1 = parts[1, 1].reshape(N, 1)

    h0, hs = _tc_ab(x, W1, d0, d1)

    accp = _sc_aggregate(hs, row2, col2, w2)

    feat, y, rp = _tc_c(accp, h0, d0, d1,
                        b1.reshape(1, D), W_out, W_root,
                        b_out.reshape(1, 1))

    zp = _sc_scalar_agg(y.reshape(N), row3, col3)

    out2 = _tc_d(zp[0].reshape(N, 1), zp[1].reshape(N, 1), y, rp,
                 c0, c1)
    return out2.reshape(-1), feat


# speedup vs baseline: 10.2331x; 1.0048x over previous
import functools

import jax
import jax.numpy as jnp
from jax import lax
from jax.experimental import pallas as pl
from jax.experimental.pallas import tpu as pltpu
from jax.experimental.pallas import tpu_sc as plsc

N = 10000
E = 320000
D = 128
HD = D // 2
NC = 2
NS = 16
NWK = NC * NS
K = 128
CS = 8

EP = 327680
W1N = EP // NWK // K
C1N = W1N // CS
W2N = EP // NS // K
C2N = W2N // CS

NZC = N // K
NZR = N - NZC * K

_mesh = plsc.VectorSubcoreMesh(core_axis_name="c", subcore_axis_name="s",
                               num_cores=NC, num_subcores=NS)
f32 = jnp.float32


def _zero16():
    return jnp.zeros((16,), f32)


@functools.partial(
    pl.kernel,
    out_type=jax.ShapeDtypeStruct((NC, 2, N), f32),
    mesh=_mesh,
    scratch_types=[
        pltpu.VMEM((CS, K), jnp.int32),
        pltpu.VMEM((CS, K), jnp.int32),
        pltpu.VMEM((CS, K), f32),
        pltpu.VMEM((CS, K), f32),
        pltpu.VMEM((2000,), f32),
        pltpu.VMEM_SHARED((N,), f32),
        pltpu.VMEM_SHARED((N,), f32),
    ],
)
def _sc_degrees(row3, col3, w3, parts, row_v, col_v, w_v, val_v, zz,
                deg_sp, cnt_sp):
    c = lax.axis_index("c")
    s = lax.axis_index("s")
    wid = s * NC + c

    @pl.when(s == 0)
    def _zero():
        def zb(i, _):
            zz[pl.ds(i * 16, 16)] = _zero16()
            return ()
        lax.fori_loop(0, 125, zb, ())

        def cb(j, _):
            pltpu.sync_copy(zz, deg_sp.at[pl.ds(j * 2000, 2000)])
            pltpu.sync_copy(zz, cnt_sp.at[pl.ds(j * 2000, 2000)])
            return ()
        lax.fori_loop(0, 5, cb, ())

    plsc.subcore_barrier()

    one = jnp.ones((16,), f32)
    zero = _zero16()

    def cb(ch, _):
        pltpu.sync_copy(row3.at[wid, pl.ds(ch * CS, CS)], row_v)
        pltpu.sync_copy(col3.at[wid, pl.ds(ch * CS, CS)], col_v)
        pltpu.sync_copy(w3.at[wid, pl.ds(ch * CS, CS)], w_v)

        def wb(w8, _):
            for j in range(K // 16):
                rv = row_v[w8, pl.ds(j * 16, 16)]
                cv = col_v[w8, pl.ds(j * 16, 16)]
                val_v[w8, pl.ds(j * 16, 16)] = jnp.where(rv != cv, one, zero)
            pltpu.sync_copy(w_v.at[w8], deg_sp.at[col_v.at[w8]], add=True)
            pltpu.sync_copy(val_v.at[w8], cnt_sp.at[col_v.at[w8]], add=True)
            return ()
        lax.fori_loop(0, CS, wb, ())
        return ()
    lax.fori_loop(0, C1N, cb, ())

    plsc.subcore_barrier()

    @pl.when(s == 0)
    def _out():
        pltpu.sync_copy(deg_sp, parts.at[c, 0])
        pltpu.sync_copy(cnt_sp, parts.at[c, 1])


HN = N // NC
AR = HN + 8
AZC = AR // K
AZR = AR - AZC * K
ADR = HN - AZC * K

_AGG_KW = dict(
    out_type=jax.ShapeDtypeStruct((N, D), f32),
    mesh=_mesh,
    scratch_types=[
        pltpu.VMEM((CS, K), jnp.int32),
        pltpu.VMEM((CS, K), jnp.int32),
        pltpu.VMEM((CS, K), f32),
        pltpu.VMEM((K, D), f32),
        pltpu.VMEM((K, D), f32),
        pltpu.VMEM((K, D), f32),
        pltpu.VMEM((K, D), f32),
        pltpu.VMEM_SHARED((AR, D), f32),
        pltpu.SemaphoreType.DMA,
        pltpu.SemaphoreType.DMA,
        pltpu.SemaphoreType.DMA,
        pltpu.SemaphoreType.DMA,
        pltpu.SemaphoreType.DMA,
        pltpu.SemaphoreType.DMA,
        pltpu.SemaphoreType.DMA,
        pltpu.SemaphoreType.DMA,
    ],
)


def _sc_aggregate_body(hs, row2, col2, w2, accp, row_v, col_v, w_v,
                       b0, b1, b2, b3, acc_sp,
                       g0, g1, g2, g3, s0, s1, s2, s3):
    c = lax.axis_index("c")
    s = lax.axis_index("s")

    def zb(i, _):
        for j in range(D // 16):
            b0[i, pl.ds(j * 16, 16)] = _zero16()
        return ()
    lax.fori_loop(0, K, zb, ())

    def zc(k, _):
        cid = k * NS + s

        @pl.when(cid < AZC)
        def _full():
            pltpu.sync_copy(b0, acc_sp.at[pl.ds(cid * K, K)])

        @pl.when(cid == AZC)
        def _rem():
            pltpu.sync_copy(b0.at[pl.ds(0, AZR)],
                            acc_sp.at[pl.ds(AZC * K, AZR)])
        return ()
    lax.fori_loop(0, 3, zc, ())

    plsc.subcore_barrier()

    nbase = c * HN

    def scale(w8, rb):
        def gb(g, _):
            wv = w_v[w8, pl.ds(g * 16, 16)]
            for l in range(16):
                r = g * 16 + l
                spl = jnp.broadcast_to(wv[l], (16,))
                for cc in range(D // 16):
                    rb[r, pl.ds(cc * 16, 16)] = (
                        rb[r, pl.ds(cc * 16, 16)] * spl)
            return ()
        lax.fori_loop(0, K // 16, gb, ())

    BUFS = [b0, b1, b2, b3]
    GS = [g0, g1, g2, g3]
    SS = [s0, s1, s2, s3]

    def cb(ch, _):
        pltpu.sync_copy(row2.at[s, pl.ds(ch * CS, CS)], row_v)
        pltpu.sync_copy(col2.at[s, pl.ds(ch * CS, CS)], col_v)
        pltpu.sync_copy(w2.at[s, pl.ds(ch * CS, CS)], w_v)

        def lb(w8, _):
            for j in range(K // 16):
                sl = pl.ds(j * 16, 16)
                loc = col_v[w8, sl] - nbase
                oob = (loc < 0) | (loc >= HN)
                col_v[w8, sl] = jnp.where(oob, HN, loc)
            return ()
        lax.fori_loop(0, CS, lb, ())

        pltpu.async_copy(hs.at[row_v.at[0]], BUFS[0], GS[0])
        pltpu.async_copy(hs.at[row_v.at[1]], BUFS[1], GS[1])
        for w in range(CS):
            b = w % 4
            if w < 2:
                gb_ = (w + 2) % 4
                pltpu.async_copy(hs.at[row_v.at[w + 2]], BUFS[gb_],
                                 GS[gb_])
            elif w <= 5:
                gb_ = (w + 2) % 4
                pltpu.make_async_copy(BUFS[gb_],
                                      acc_sp.at[col_v.at[w - 2]],
                                      SS[gb_]).wait()
                pltpu.async_copy(hs.at[row_v.at[w + 2]], BUFS[gb_],
                                 GS[gb_])
            pltpu.make_async_copy(hs.at[row_v.at[w]], BUFS[b],
                                  GS[b]).wait()
            scale(w, BUFS[b])
            pltpu.async_copy(BUFS[b], acc_sp.at[col_v.at[w]], SS[b],
                             add=True)
        for w in range(CS - 4, CS):
            b = w % 4
            pltpu.make_async_copy(BUFS[b], acc_sp.at[col_v.at[w]],
                                  SS[b]).wait()
        return ()
    lax.fori_loop(0, C2N, cb, ())

    plsc.subcore_barrier()

    def oc(k, _):
        cid = k * NS + s

        @pl.when(cid < AZC)
        def _full():
            pltpu.sync_copy(acc_sp.at[pl.ds(cid * K, K)],
                            accp.at[pl.ds(nbase + cid * K, K)])

        @pl.when(cid == AZC)
        def _rem():
            pltpu.sync_copy(acc_sp.at[pl.ds(AZC * K, ADR)],
                            accp.at[pl.ds(nbase + AZC * K, ADR)])
        return ()
    lax.fori_loop(0, 3, oc, ())


_sc_aggregate = pl.kernel(_sc_aggregate_body, **_AGG_KW)


@functools.partial(
    pl.kernel,
    out_type=jax.ShapeDtypeStruct((NC, N), f32),
    mesh=_mesh,
    scratch_types=[
        pltpu.VMEM((CS, K), jnp.int32),
        pltpu.VMEM((CS, K), jnp.int32),
        pltpu.VMEM((K,), f32),
        pltpu.VMEM((CS, K), f32),
        pltpu.VMEM((2000,), f32),
        pltpu.VMEM_SHARED((N,), f32),
        pltpu.SemaphoreType.DMA,
    ],
)
def _sc_scalar_agg(y1, row3, col3, zp, row_v, col_v, yv_b, val_v, zz,
                   z_sp, sem):
    c = lax.axis_index("c")
    s = lax.axis_index("s")
    wid = s * NC + c

    @pl.when(s == 0)
    def _zero():
        def zb(i, _):
            zz[pl.ds(i * 16, 16)] = _zero16()
            return ()
        lax.fori_loop(0, 125, zb, ())

        def cb(j, _):
            pltpu.sync_copy(zz, z_sp.at[pl.ds(j * 2000, 2000)])
            return ()
        lax.fori_loop(0, 5, cb, ())

    plsc.subcore_barrier()

    zero = _zero16()

    def cb(ch, _):
        pltpu.sync_copy(row3.at[wid, pl.ds(ch * CS, CS)], row_v)
        pltpu.sync_copy(col3.at[wid, pl.ds(ch * CS, CS)], col_v)

        def wb(w8, _):
            pltpu.async_copy(y1.at[row_v.at[w8]], yv_b, sem).wait()
            for j in range(K // 16):
                rv = row_v[w8, pl.ds(j * 16, 16)]
                cv = col_v[w8, pl.ds(j * 16, 16)]
                yv = yv_b[pl.ds(j * 16, 16)]
                val_v[w8, pl.ds(j * 16, 16)] = jnp.where(rv != cv, yv, zero)
            pltpu.sync_copy(val_v.at[w8], z_sp.at[col_v.at[w8]], add=True)
            return ()
        lax.fori_loop(0, CS, wb, ())
        return ()
    lax.fori_loop(0, C1N, cb, ())

    plsc.subcore_barrier()

    @pl.when(s == 0)
    def _out():
        pltpu.sync_copy(z_sp, zp.at[c])


RB = 2000
GRID = N // RB


def _tc_ab_body(x_r, w1_r, d0_r, d1_r, h0_r, hs_r):
    h0 = jnp.dot(x_r[...], w1_r[...], preferred_element_type=f32)
    deg = d0_r[...] + d1_r[...] + 1.0
    h0_r[...] = h0
    hs_r[...] = h0 * lax.rsqrt(deg)


def _tc_ab(x, w1, d0, d1):
    return pl.pallas_call(
        _tc_ab_body,
        grid=(GRID,),
        in_specs=[
            pl.BlockSpec((RB, D), lambda i: (i, 0)),
            pl.BlockSpec((D, D), lambda i: (0, 0)),
            pl.BlockSpec((RB, 1), lambda i: (i, 0)),
            pl.BlockSpec((RB, 1), lambda i: (i, 0)),
        ],
        out_specs=[
            pl.BlockSpec((RB, D), lambda i: (i, 0)),
            pl.BlockSpec((RB, D), lambda i: (i, 0)),
        ],
        out_shape=[
            jax.ShapeDtypeStruct((N, D), f32),
            jax.ShapeDtypeStruct((N, D), f32),
        ],
    )(x, w1, d0, d1)


def _tc_c_body(acc_r, h0_r, d0_r, d1_r, b1_r, wo_r, wr_r, bo_r,
               feat_r, y_r, rp_r):
    deg = d0_r[...] + d1_r[...] + 1.0
    dis = lax.rsqrt(deg)
    h = dis * acc_r[...] + h0_r[...] / deg + b1_r[...]
    h = jnp.maximum(h, 0.0)
    feat_r[...] = h
    y_r[...] = jnp.dot(h, wo_r[...], preferred_element_type=f32)
    rp_r[...] = (jnp.dot(h, wr_r[...], preferred_element_type=f32)
                 + bo_r[...])


def _tc_c(accp, h0, d0, d1, b1, wo, wr, bo):
    return pl.pallas_call(
        _tc_c_body,
        grid=(GRID,),
        in_specs=[
            pl.BlockSpec((RB, D), lambda i: (i, 0)),
            pl.BlockSpec((RB, D), lambda i: (i, 0)),
            pl.BlockSpec((RB, 1), lambda i: (i, 0)),
            pl.BlockSpec((RB, 1), lambda i: (i, 0)),
            pl.BlockSpec((1, D), lambda i: (0, 0)),
            pl.BlockSpec((D, 1), lambda i: (0, 0)),
            pl.BlockSpec((D, 1), lambda i: (0, 0)),
            pl.BlockSpec((1, 1), lambda i: (0, 0)),
        ],
        out_specs=[
            pl.BlockSpec((RB, D), lambda i: (i, 0)),
            pl.BlockSpec((RB, 1), lambda i: (i, 0)),
            pl.BlockSpec((RB, 1), lambda i: (i, 0)),
        ],
        out_shape=[
            jax.ShapeDtypeStruct((N, D), f32),
            jax.ShapeDtypeStruct((N, 1), f32),
            jax.ShapeDtypeStruct((N, 1), f32),
        ],
    )(accp, h0, d0, d1, b1, wo, wr, bo)


def _tc_d_body(z0_r, z1_r, y_r, rp_r, c0_r, c1_r, o_r):
    cnt = c0_r[...] + c1_r[...] + 1.0
    o_r[...] = (z0_r[...] + z1_r[...] + y_r[...]) / jnp.maximum(cnt, 1.0) \
        + rp_r[...]


def _tc_d(z0, z1, y, rp, c0, c1):
    return pl.pallas_call(
        _tc_d_body,
        grid=(GRID,),
        in_specs=[pl.BlockSpec((RB, 1), lambda i: (i, 0))] * 6,
        out_specs=pl.BlockSpec((RB, 1), lambda i: (i, 0)),
        out_shape=jax.ShapeDtypeStruct((N, 1), f32),
    )(z0, z1, y, rp, c0, c1)


def kernel(x, edge_index, edge_weight, W1, b1, W_out, b_out, W_root):
    pad = EP - E
    rowp = jnp.concatenate([edge_index[0],
                            jnp.zeros((pad,), jnp.int32)])
    colp = jnp.concatenate([edge_index[1],
                            jnp.zeros((pad,), jnp.int32)])
    wp = jnp.concatenate([edge_weight, jnp.zeros((pad,), f32)])

    row3 = rowp.reshape(NWK, W1N, K)
    col3 = colp.reshape(NWK, W1N, K)
    w3 = wp.reshape(NWK, W1N, K)
    row2 = rowp.reshape(NS, W2N, K)
    col2 = colp.reshape(NS, W2N, K)
    w2 = wp.reshape(NS, W2N, K)

    parts = _sc_degrees(row3, col3, w3)
    d0 = parts[0, 0].reshape(N, 1)
    d1 = parts[1, 0].reshape(N, 1)
    c0 = parts[0, 1].reshape(N, 1)
    c1 = parts[1, 1].reshape(N, 1)

    h0, hs = _tc_ab(x, W1, d0, d1)

    accp = _sc_aggregate(hs, row2, col2, w2)

    feat, y, rp = _tc_c(accp, h0, d0, d1,
                        b1.reshape(1, D), W_out, W_root,
                        b_out.reshape(1, 1))

    zp = _sc_scalar_agg(y.reshape(N), row3, col3)

    out2 = _tc_d(zp[0].reshape(N, 1), zp[1].reshape(N, 1), y, rp,
                 c0, c1)
    return out2.reshape(-1), feat


# speedup vs baseline: 10.2501x; 1.0017x over previous
import functools

import jax
import jax.numpy as jnp
from jax import lax
from jax.experimental import pallas as pl
from jax.experimental.pallas import tpu as pltpu
from jax.experimental.pallas import tpu_sc as plsc

N = 10000
E = 320000
D = 128
HD = D // 2
NC = 2
NS = 16
NWK = NC * NS
K = 128
CS = 8

EP = 327680
W1N = EP // NWK // K
C1N = W1N // CS
W2N = EP // NS // K
CS2 = 16
C2N = W2N // CS2

NZC = N // K
NZR = N - NZC * K

_mesh = plsc.VectorSubcoreMesh(core_axis_name="c", subcore_axis_name="s",
                               num_cores=NC, num_subcores=NS)
f32 = jnp.float32


def _zero16():
    return jnp.zeros((16,), f32)


@functools.partial(
    pl.kernel,
    out_type=jax.ShapeDtypeStruct((NC, 2, N), f32),
    mesh=_mesh,
    scratch_types=[
        pltpu.VMEM((CS, K), jnp.int32),
        pltpu.VMEM((CS, K), jnp.int32),
        pltpu.VMEM((CS, K), f32),
        pltpu.VMEM((CS, K), f32),
        pltpu.VMEM((2000,), f32),
        pltpu.VMEM_SHARED((N,), f32),
        pltpu.VMEM_SHARED((N,), f32),
    ],
)
def _sc_degrees(row3, col3, w3, parts, row_v, col_v, w_v, val_v, zz,
                deg_sp, cnt_sp):
    c = lax.axis_index("c")
    s = lax.axis_index("s")
    wid = s * NC + c

    @pl.when(s == 0)
    def _zero():
        def zb(i, _):
            zz[pl.ds(i * 16, 16)] = _zero16()
            return ()
        lax.fori_loop(0, 125, zb, ())

        def cb(j, _):
            pltpu.sync_copy(zz, deg_sp.at[pl.ds(j * 2000, 2000)])
            pltpu.sync_copy(zz, cnt_sp.at[pl.ds(j * 2000, 2000)])
            return ()
        lax.fori_loop(0, 5, cb, ())

    plsc.subcore_barrier()

    one = jnp.ones((16,), f32)
    zero = _zero16()

    def cb(ch, _):
        pltpu.sync_copy(row3.at[wid, pl.ds(ch * CS, CS)], row_v)
        pltpu.sync_copy(col3.at[wid, pl.ds(ch * CS, CS)], col_v)
        pltpu.sync_copy(w3.at[wid, pl.ds(ch * CS, CS)], w_v)

        def wb(w8, _):
            for j in range(K // 16):
                rv = row_v[w8, pl.ds(j * 16, 16)]
                cv = col_v[w8, pl.ds(j * 16, 16)]
                val_v[w8, pl.ds(j * 16, 16)] = jnp.where(rv != cv, one, zero)
            pltpu.sync_copy(w_v.at[w8], deg_sp.at[col_v.at[w8]], add=True)
            pltpu.sync_copy(val_v.at[w8], cnt_sp.at[col_v.at[w8]], add=True)
            return ()
        lax.fori_loop(0, CS, wb, ())
        return ()
    lax.fori_loop(0, C1N, cb, ())

    plsc.subcore_barrier()

    @pl.when(s == 0)
    def _out():
        pltpu.sync_copy(deg_sp, parts.at[c, 0])
        pltpu.sync_copy(cnt_sp, parts.at[c, 1])


HN = N // NC
AR = HN + 8
AZC = AR // K
AZR = AR - AZC * K
ADR = HN - AZC * K

_AGG_KW = dict(
    out_type=jax.ShapeDtypeStruct((N, D), f32),
    mesh=_mesh,
    scratch_types=[
        pltpu.VMEM((CS2, K), jnp.int32),
        pltpu.VMEM((CS2, K), jnp.int32),
        pltpu.VMEM((CS2, K), f32),
        pltpu.VMEM((K, D), f32),
        pltpu.VMEM((K, D), f32),
        pltpu.VMEM((K, D), f32),
        pltpu.VMEM((K, D), f32),
        pltpu.VMEM_SHARED((AR, D), f32),
        pltpu.SemaphoreType.DMA,
        pltpu.SemaphoreType.DMA,
        pltpu.SemaphoreType.DMA,
        pltpu.SemaphoreType.DMA,
        pltpu.SemaphoreType.DMA,
        pltpu.SemaphoreType.DMA,
        pltpu.SemaphoreType.DMA,
        pltpu.SemaphoreType.DMA,
    ],
)


def _sc_aggregate_body(hs, row2, col2, w2, accp, row_v, col_v, w_v,
                       b0, b1, b2, b3, acc_sp,
                       g0, g1, g2, g3, s0, s1, s2, s3):
    c = lax.axis_index("c")
    s = lax.axis_index("s")

    def zb(i, _):
        for j in range(D // 16):
            b0[i, pl.ds(j * 16, 16)] = _zero16()
        return ()
    lax.fori_loop(0, K, zb, ())

    def zc(k, _):
        cid = k * NS + s

        @pl.when(cid < AZC)
        def _full():
            pltpu.sync_copy(b0, acc_sp.at[pl.ds(cid * K, K)])

        @pl.when(cid == AZC)
        def _rem():
            pltpu.sync_copy(b0.at[pl.ds(0, AZR)],
                            acc_sp.at[pl.ds(AZC * K, AZR)])
        return ()
    lax.fori_loop(0, 3, zc, ())

    plsc.subcore_barrier()

    nbase = c * HN

    def scale(w8, rb):
        def gb(g, _):
            wv = w_v[w8, pl.ds(g * 16, 16)]
            for l in range(16):
                r = g * 16 + l
                spl = jnp.broadcast_to(wv[l], (16,))
                for cc in range(D // 16):
                    rb[r, pl.ds(cc * 16, 16)] = (
                        rb[r, pl.ds(cc * 16, 16)] * spl)
            return ()
        lax.fori_loop(0, K // 16, gb, ())

    BUFS = [b0, b1, b2, b3]
    GS = [g0, g1, g2, g3]
    SS = [s0, s1, s2, s3]

    def cb(ch, _):
        pltpu.sync_copy(row2.at[s, pl.ds(ch * CS2, CS2)], row_v)
        pltpu.sync_copy(col2.at[s, pl.ds(ch * CS2, CS2)], col_v)
        pltpu.sync_copy(w2.at[s, pl.ds(ch * CS2, CS2)], w_v)

        def lb(w8, _):
            for j in range(K // 16):
                sl = pl.ds(j * 16, 16)
                loc = col_v[w8, sl] - nbase
                oob = (loc < 0) | (loc >= HN)
                col_v[w8, sl] = jnp.where(oob, HN, loc)
            return ()
        lax.fori_loop(0, CS2, lb, ())

        pltpu.async_copy(hs.at[row_v.at[0]], BUFS[0], GS[0])
        pltpu.async_copy(hs.at[row_v.at[1]], BUFS[1], GS[1])

        def ql(q, _):
            for i in range(4):
                w = 4 * q + i
                b = i
                pw = w + 2
                pb_ = (i + 2) % 4

                @pl.when(pw < CS2)
                def _pf():
                    @pl.when(q + (1 if i >= 2 else 0) > 0)
                    def _wsct():
                        pltpu.make_async_copy(
                            BUFS[pb_], acc_sp.at[col_v.at[w - 2]],
                            SS[pb_]).wait()
                    pltpu.async_copy(hs.at[row_v.at[pw]], BUFS[pb_],
                                    GS[pb_])
                pltpu.make_async_copy(hs.at[row_v.at[w]], BUFS[b],
                                      GS[b]).wait()
                scale(w, BUFS[b])
                pltpu.async_copy(BUFS[b], acc_sp.at[col_v.at[w]],
                                 SS[b], add=True)
            return ()
        lax.fori_loop(0, CS2 // 4, ql, ())
        for w in range(CS2 - 4, CS2):
            b = w % 4
            pltpu.make_async_copy(BUFS[b], acc_sp.at[col_v.at[w]],
                                  SS[b]).wait()
        return ()
    lax.fori_loop(0, C2N, cb, ())

    plsc.subcore_barrier()

    def oc(k, _):
        cid = k * NS + s

        @pl.when(cid < AZC)
        def _full():
            pltpu.sync_copy(acc_sp.at[pl.ds(cid * K, K)],
                            accp.at[pl.ds(nbase + cid * K, K)])

        @pl.when(cid == AZC)
        def _rem():
            pltpu.sync_copy(acc_sp.at[pl.ds(AZC * K, ADR)],
                            accp.at[pl.ds(nbase + AZC * K, ADR)])
        return ()
    lax.fori_loop(0, 3, oc, ())


_sc_aggregate = pl.kernel(_sc_aggregate_body, **_AGG_KW)


@functools.partial(
    pl.kernel,
    out_type=jax.ShapeDtypeStruct((NC, N), f32),
    mesh=_mesh,
    scratch_types=[
        pltpu.VMEM((CS, K), jnp.int32),
        pltpu.VMEM((CS, K), jnp.int32),
        pltpu.VMEM((K,), f32),
        pltpu.VMEM((CS, K), f32),
        pltpu.VMEM((2000,), f32),
        pltpu.VMEM_SHARED((N,), f32),
        pltpu.SemaphoreType.DMA,
    ],
)
def _sc_scalar_agg(y1, row3, col3, zp, row_v, col_v, yv_b, val_v, zz,
                   z_sp, sem):
    c = lax.axis_index("c")
    s = lax.axis_index("s")
    wid = s * NC + c

    @pl.when(s == 0)
    def _zero():
        def zb(i, _):
            zz[pl.ds(i * 16, 16)] = _zero16()
            return ()
        lax.fori_loop(0, 125, zb, ())

        def cb(j, _):
            pltpu.sync_copy(zz, z_sp.at[pl.ds(j * 2000, 2000)])
            return ()
        lax.fori_loop(0, 5, cb, ())

    plsc.subcore_barrier()

    zero = _zero16()

    def cb(ch, _):
        pltpu.sync_copy(row3.at[wid, pl.ds(ch * CS, CS)], row_v)
        pltpu.sync_copy(col3.at[wid, pl.ds(ch * CS, CS)], col_v)

        def wb(w8, _):
            pltpu.async_copy(y1.at[row_v.at[w8]], yv_b, sem).wait()
            for j in range(K // 16):
                rv = row_v[w8, pl.ds(j * 16, 16)]
                cv = col_v[w8, pl.ds(j * 16, 16)]
                yv = yv_b[pl.ds(j * 16, 16)]
                val_v[w8, pl.ds(j * 16, 16)] = jnp.where(rv != cv, yv, zero)
            pltpu.sync_copy(val_v.at[w8], z_sp.at[col_v.at[w8]], add=True)
            return ()
        lax.fori_loop(0, CS, wb, ())
        return ()
    lax.fori_loop(0, C1N, cb, ())

    plsc.subcore_barrier()

    @pl.when(s == 0)
    def _out():
        pltpu.sync_copy(z_sp, zp.at[c])


RB = 2000
GRID = N // RB


def _tc_ab_body(x_r, w1_r, d0_r, d1_r, h0_r, hs_r):
    h0 = jnp.dot(x_r[...], w1_r[...], preferred_element_type=f32)
    deg = d0_r[...] + d1_r[...] + 1.0
    h0_r[...] = h0
    hs_r[...] = h0 * lax.rsqrt(deg)


def _tc_ab(x, w1, d0, d1):
    return pl.pallas_call(
        _tc_ab_body,
        grid=(GRID,),
        in_specs=[
            pl.BlockSpec((RB, D), lambda i: (i, 0)),
            pl.BlockSpec((D, D), lambda i: (0, 0)),
            pl.BlockSpec((RB, 1), lambda i: (i, 0)),
            pl.BlockSpec((RB, 1), lambda i: (i, 0)),
        ],
        out_specs=[
            pl.BlockSpec((RB, D), lambda i: (i, 0)),
            pl.BlockSpec((RB, D), lambda i: (i, 0)),
        ],
        out_shape=[
            jax.ShapeDtypeStruct((N, D), f32),
            jax.ShapeDtypeStruct((N, D), f32),
        ],
    )(x, w1, d0, d1)


def _tc_c_body(acc_r, h0_r, d0_r, d1_r, b1_r, wo_r, wr_r, bo_r,
               feat_r, y_r, rp_r):
    deg = d0_r[...] + d1_r[...] + 1.0
    dis = lax.rsqrt(deg)
    h = dis * acc_r[...] + h0_r[...] / deg + b1_r[...]
    h = jnp.maximum(h, 0.0)
    feat_r[...] = h
    y_r[...] = jnp.dot(h, wo_r[...], preferred_element_type=f32)
    rp_r[...] = (jnp.dot(h, wr_r[...], preferred_element_type=f32)
                 + bo_r[...])


def _tc_c(accp, h0, d0, d1, b1, wo, wr, bo):
    return pl.pallas_call(
        _tc_c_body,
        grid=(GRID,),
        in_specs=[
            pl.BlockSpec((RB, D), lambda i: (i, 0)),
            pl.BlockSpec((RB, D), lambda i: (i, 0)),
            pl.BlockSpec((RB, 1), lambda i: (i, 0)),
            pl.BlockSpec((RB, 1), lambda i: (i, 0)),
            pl.BlockSpec((1, D), lambda i: (0, 0)),
            pl.BlockSpec((D, 1), lambda i: (0, 0)),
            pl.BlockSpec((D, 1), lambda i: (0, 0)),
            pl.BlockSpec((1, 1), lambda i: (0, 0)),
        ],
        out_specs=[
            pl.BlockSpec((RB, D), lambda i: (i, 0)),
            pl.BlockSpec((RB, 1), lambda i: (i, 0)),
            pl.BlockSpec((RB, 1), lambda i: (i, 0)),
        ],
        out_shape=[
            jax.ShapeDtypeStruct((N, D), f32),
            jax.ShapeDtypeStruct((N, 1), f32),
            jax.ShapeDtypeStruct((N, 1), f32),
        ],
    )(accp, h0, d0, d1, b1, wo, wr, bo)


def _tc_d_body(z0_r, z1_r, y_r, rp_r, c0_r, c1_r, o_r):
    cnt = c0_r[...] + c1_r[...] + 1.0
    o_r[...] = (z0_r[...] + z1_r[...] + y_r[...]) / jnp.maximum(cnt, 1.0) \
        + rp_r[...]


def _tc_d(z0, z1, y, rp, c0, c1):
    return pl.pallas_call(
        _tc_d_body,
        grid=(GRID,),
        in_specs=[pl.BlockSpec((RB, 1), lambda i: (i, 0))] * 6,
        out_specs=pl.BlockSpec((RB, 1), lambda i: (i, 0)),
        out_shape=jax.ShapeDtypeStruct((N, 1), f32),
    )(z0, z1, y, rp, c0, c1)


def kernel(x, edge_index, edge_weight, W1, b1, W_out, b_out, W_root):
    pad = EP - E
    rowp = jnp.concatenate([edge_index[0],
                            jnp.zeros((pad,), jnp.int32)])
    colp = jnp.concatenate([edge_index[1],
                            jnp.zeros((pad,), jnp.int32)])
    wp = jnp.concatenate([edge_weight, jnp.zeros((pad,), f32)])

    row3 = rowp.reshape(NWK, W1N, K)
    col3 = colp.reshape(NWK, W1N, K)
    w3 = wp.reshape(NWK, W1N, K)
    row2 = rowp.reshape(NS, W2N, K)
    col2 = colp.reshape(NS, W2N, K)
    w2 = wp.reshape(NS, W2N, K)

    parts = _sc_degrees(row3, col3, w3)
    d0 = parts[0, 0].reshape(N, 1)
    d1 = parts[1, 0].reshape(N, 1)
    c0 = parts[0, 1].reshape(N, 1)
    c1 = parts[1, 1].reshape(N, 1)

    h0, hs = _tc_ab(x, W1, d0, d1)

    accp = _sc_aggregate(hs, row2, col2, w2)

    feat, y, rp = _tc_c(accp, h0, d0, d1,
                        b1.reshape(1, D), W_out, W_root,
                        b_out.reshape(1, 1))

    zp = _sc_scalar_agg(y.reshape(N), row3, col3)

    out2 = _tc_d(zp[0].reshape(N, 1), zp[1].reshape(N, 1), y, rp,
                 c0, c1)
    return out2.reshape(-1), feat
